# baseline (speedup 1.0000x reference)
import functools

import jax
import jax.numpy as jnp
from jax import lax
from jax.experimental import pallas as pl
from jax.experimental.pallas import tpu as pltpu
from jax.experimental.pallas import tpu_sc as plsc

D = 2048
_NC, _NS = 2, 16
_NW = _NC * _NS
_CHUNK = 32


def _gather_body(table_hbm, idx_hbm, out_hbm, idx_v, rows_v, sem):
    rows_w = idx_hbm.shape[0] // _NW
    wid = lax.axis_index("s") * _NC + lax.axis_index("c")
    base = wid * rows_w
    pltpu.sync_copy(idx_hbm.at[pl.ds(base, rows_w)], idx_v)

    def chunk(c, carry):
        r0 = c * _CHUNK
        pltpu.async_copy(
            table_hbm.at[idx_v.at[pl.ds(r0, _CHUNK)]], rows_v, sem
        ).wait()
        pltpu.sync_copy(rows_v, out_hbm.at[pl.ds(base + r0, _CHUNK)])
        return carry

    lax.fori_loop(0, rows_w // _CHUNK, chunk, 0)


def _sc_gather(table, idx):
    n = idx.shape[0]
    rows_w = n // _NW
    mesh = plsc.VectorSubcoreMesh(core_axis_name="c", subcore_axis_name="s")
    return pl.kernel(
        _gather_body,
        out_type=jax.ShapeDtypeStruct((n, D), jnp.float32),
        mesh=mesh,
        scratch_types=[
            pltpu.VMEM((rows_w,), jnp.int32),
            pltpu.VMEM((_CHUNK, D), jnp.float32),
            pltpu.SemaphoreType.DMA,
        ],
    )(table, idx)


def _mm_body(x_ref, m_ref, w0_ref, w1_ref, o_ref):
    x = x_ref[...].astype(jnp.bfloat16)
    dn = (((1,), (1,)), ((), ()))
    y0 = lax.dot_general(x, w0_ref[...], dn, preferred_element_type=jnp.float32)
    y1 = lax.dot_general(x, w1_ref[...], dn, preferred_element_type=jnp.float32)
    o_ref[...] = jnp.where(m_ref[...] == 0, y0, y1)


def _tc_matmul_select(x, m, w0, w1, bn=256):
    n = x.shape[0]
    grid = (n // bn,)
    return pl.pallas_call(
        _mm_body,
        grid=grid,
        in_specs=[
            pl.BlockSpec((bn, D), lambda i: (i, 0)),
            pl.BlockSpec((bn, 1), lambda i: (i, 0)),
            pl.BlockSpec((D, D), lambda i: (0, 0)),
            pl.BlockSpec((D, D), lambda i: (0, 0)),
        ],
        out_specs=pl.BlockSpec((bn, D), lambda i: (i, 0)),
        out_shape=jax.ShapeDtypeStruct((n, D), jnp.float32),
    )(x, m, w0, w1)


def kernel(input_ids, role_mask, table, W0, W1):
    b, l = input_ids.shape
    n = b * l
    idx = input_ids.reshape(n).astype(jnp.int32)
    m = role_mask.reshape(n, 1).astype(jnp.int32)
    w0b = W0.astype(jnp.bfloat16)
    w1b = W1.astype(jnp.bfloat16)
    s = 4
    ns = n // s
    ys = []
    for k in range(s):
        xk = _sc_gather(table, lax.slice(idx, (k * ns,), ((k + 1) * ns,)))
        mk = lax.slice(m, (k * ns, 0), ((k + 1) * ns, 1))
        ys.append(_tc_matmul_select(xk, mk, w0b, w1b))
    return jnp.concatenate(ys, axis=0).reshape(b, l, D)


# speedup vs baseline: 1.2249x; 1.2249x over previous
import functools

import jax
import jax.numpy as jnp
from jax import lax
from jax.experimental import pallas as pl
from jax.experimental.pallas import tpu as pltpu
from jax.experimental.pallas import tpu_sc as plsc

D = 2048
_NC, _NS = 2, 16
_NW = _NC * _NS
_CHUNK = 32


def _gather_body(table_hbm, idx_hbm, out_hbm, idx_v, rows_v, sem):
    rows_w = idx_hbm.shape[0] // _NW
    wid = lax.axis_index("s") * _NC + lax.axis_index("c")
    base = wid * rows_w
    pltpu.sync_copy(idx_hbm.at[pl.ds(base, rows_w)], idx_v)

    def chunk(c, carry):
        r0 = c * _CHUNK
        pltpu.async_copy(
            table_hbm.at[idx_v.at[pl.ds(r0, _CHUNK)]], rows_v, sem
        ).wait()
        pltpu.sync_copy(rows_v, out_hbm.at[pl.ds(base + r0, _CHUNK)])
        return carry

    lax.fori_loop(0, rows_w // _CHUNK, chunk, 0)


def _sc_gather(table, idx):
    n = idx.shape[0]
    rows_w = n // _NW
    mesh = plsc.VectorSubcoreMesh(core_axis_name="c", subcore_axis_name="s")
    return pl.kernel(
        _gather_body,
        out_type=jax.ShapeDtypeStruct((n, D), jnp.float32),
        mesh=mesh,
        scratch_types=[
            pltpu.VMEM((rows_w,), jnp.int32),
            pltpu.VMEM((_CHUNK, D), jnp.float32),
            pltpu.SemaphoreType.DMA,
        ],
    )(table, idx)


def _mm_body(x_ref, m_ref, w0_ref, w1_ref, o_ref):
    x = x_ref[...].astype(jnp.bfloat16)
    dn = (((1,), (1,)), ((), ()))
    y0 = lax.dot_general(x, w0_ref[...], dn, preferred_element_type=jnp.float32)
    y1 = lax.dot_general(x, w1_ref[...], dn, preferred_element_type=jnp.float32)
    o_ref[...] = jnp.where(m_ref[...] == 0, y0, y1)


def _mm_body_alias(x_ref, m_ref, w0_ref, w1_ref, _y_ref, o_ref):
    _mm_body(x_ref, m_ref, w0_ref, w1_ref, o_ref)


def _tc_matmul_select(x, m, w0, w1, n, k, y_prev, bn=256):
    ns = x.shape[0]
    base = k * (ns // bn)
    specs = [
        pl.BlockSpec((bn, D), lambda i: (i, 0)),
        pl.BlockSpec((bn, 1), lambda i: (i, 0)),
        pl.BlockSpec((D, D), lambda i: (0, 0)),
        pl.BlockSpec((D, D), lambda i: (0, 0)),
    ]
    args = [x, m, w0, w1]
    body = _mm_body
    aliases = {}
    if y_prev is not None:
        specs.append(pl.BlockSpec(memory_space=pl.ANY))
        args.append(y_prev)
        body = _mm_body_alias
        aliases = {4: 0}
    return pl.pallas_call(
        body,
        grid=(ns // bn,),
        in_specs=specs,
        out_specs=pl.BlockSpec((bn, D), lambda i: (base + i, 0)),
        out_shape=jax.ShapeDtypeStruct((n, D), jnp.float32),
        input_output_aliases=aliases,
    )(*args)


def kernel(input_ids, role_mask, table, W0, W1):
    b, l = input_ids.shape
    n = b * l
    idx = input_ids.reshape(n).astype(jnp.int32)
    m = role_mask.reshape(n, 1).astype(jnp.int32)
    w0b = W0.astype(jnp.bfloat16)
    w1b = W1.astype(jnp.bfloat16)
    s = 4
    ns = n // s
    y = None
    for k in range(s):
        xk = _sc_gather(table, lax.slice(idx, (k * ns,), ((k + 1) * ns,)))
        mk = lax.slice(m, (k * ns, 0), ((k + 1) * ns, 1))
        y = _tc_matmul_select(xk, mk, w0b, w1b, n, k, y)
    return y.reshape(b, l, D)
